# baseline (speedup 1.0000x reference)
import jax
import jax.numpy as jnp
from jax import lax
from jax.experimental import pallas as pl
from jax.experimental.pallas import tpu as pltpu
from jax.experimental.pallas import tpu_sc as plsc

_ACTIONS = 100
_APAD = 104
_EMB = 32
_DELTA = 5000
_LANES = 16

_NC = 2
_NS = 16
_NW = _NC * _NS
_BLK = 128


def _permute_index(i):
    return ((i >> 12) * 4096 + (i & 1023) * 4 + ((i >> 10) & 3))


def _convert_body(bt_ref, out_ref):
    y = bt_ref[...]
    zs = [y[:, j * 1024:(j + 1) * 1024].T for j in range(4)]
    out_ref[...] = jnp.concatenate(zs, axis=1)


def _prelude_body(x_ref, gamma_ref, beta_ref, idxlo_ref, idxhi_ref,
                  wl_ref, wh_ref):
    x = x_ref[...]
    mean = jnp.mean(x, axis=1, keepdims=True)
    var = jnp.mean((x - mean) ** 2, axis=1, keepdims=True)
    xn = (x - mean) / jnp.sqrt(var + 1e-5) * gamma_ref[...] + beta_ref[...]
    xt = jnp.tanh(xn)
    xc = jnp.clip(xt, -1.0 + 1e-5, 1.0 - 1e-5)
    ind = lax.broadcasted_iota(jnp.int32, x.shape, 0)
    xl = jnp.floor(xc * _DELTA)
    xh = jnp.floor(xc * _DELTA + 1)
    xli = _permute_index(_ACTIONS * (xl.astype(jnp.int32) + _DELTA) + ind)
    xhi = _permute_index(_ACTIONS * (xh.astype(jnp.int32) + _DELTA) + ind)
    d = 1.0 / _DELTA
    wh = (xc - xl / _DELTA) / d
    wl = (xh / _DELTA - xc) / d
    pad = _APAD - _ACTIONS
    zi = jnp.zeros((pad, x.shape[1]), jnp.int32)
    zf = jnp.zeros((pad, x.shape[1]), jnp.float32)
    idxlo_ref[...] = jnp.concatenate([xli, zi], axis=0)
    idxhi_ref[...] = jnp.concatenate([xhi, zi], axis=0)
    wl_ref[...] = jnp.concatenate([wl, zf], axis=0)
    wh_ref[...] = jnp.concatenate([wh, zf], axis=0)


def _sc_body(idxlo_hbm, idxhi_hbm, wl_hbm, wh_hbm, table_hbm, out_hbm,
             il_v, ih_v, wl_v, wh_v, bl0_v, bl1_v, bh0_v, bh1_v,
             out0_v, out1_v, gsem0, gsem1, osem0, osem1):
    wid = lax.axis_index("s") * _NC + lax.axis_index("c")
    bs = wid * _BLK
    pltpu.sync_copy(idxlo_hbm.at[:_ACTIONS, pl.ds(bs, _BLK)], il_v)
    pltpu.sync_copy(idxhi_hbm.at[:_ACTIONS, pl.ds(bs, _BLK)], ih_v)
    pltpu.sync_copy(wl_hbm.at[:_ACTIONS, pl.ds(bs, _BLK)], wl_v)
    pltpu.sync_copy(wh_hbm.at[:_ACTIONS, pl.ds(bs, _BLK)], wh_v)

    bls = (bl0_v, bl1_v)
    bhs = (bh0_v, bh1_v)
    outs = (out0_v, out1_v)
    gsems = (gsem0, gsem1)
    osems = (osem0, osem1)

    def fire(a, slot):
        pltpu.async_copy(table_hbm.at[il_v.at[a]], bls[slot], gsems[slot])
        pltpu.async_copy(table_hbm.at[ih_v.at[a]], bhs[slot], gsems[slot])

    fire(0, 0)
    fire(1, 1)

    def body(i, carry):
        for slot in range(2):
            a = 2 * i + slot
            bl_v, bh_v, out_v = bls[slot], bhs[slot], outs[slot]
            pltpu.make_async_copy(
                table_hbm.at[il_v.at[a]], bl_v, gsems[slot]).wait()
            pltpu.make_async_copy(
                table_hbm.at[ih_v.at[a]], bh_v, gsems[slot]).wait()

            @pl.when(i > 0)
            def _():
                pltpu.make_async_copy(
                    out_v, out_hbm.at[a, :, pl.ds(bs, _BLK)],
                    osems[slot]).wait()

            ngrp = _BLK // _LANES
            base = lax.iota(jnp.int32, _LANES)

            @plsc.parallel_loop(0, _EMB * ngrp, unroll=8)
            def _(t):
                g = t & (ngrp - 1)
                e = t >> 3
                s = pl.ds(g * _LANES, _LANES)
                wlg = wl_v[a, s]
                whg = wh_v[a, s]
                rowidx = base + g * _LANES
                colidx = jnp.zeros((_LANES,), jnp.int32) + e
                blv = plsc.load_gather(bl_v, [rowidx, colidx])
                bhv = plsc.load_gather(bh_v, [rowidx, colidx])
                out_v[e, s] = blv * wlg + bhv * whg

            pltpu.async_copy(
                out_v, out_hbm.at[a, :, pl.ds(bs, _BLK)], osems[slot])

            @pl.when(i < _ACTIONS // 2 - 1)
            def _():
                fire(a + 2, slot)
        return carry

    lax.fori_loop(0, _ACTIONS // 2, body, 0)
    for slot in range(2):
        a = _ACTIONS - 2 + slot
        pltpu.make_async_copy(
            outs[slot], out_hbm.at[a, :, pl.ds(bs, _BLK)],
            osems[slot]).wait()


@jax.jit
def kernel(x, B, gamma, beta):
    n = x.shape[0]
    idxlo, idxhi, wl, wh = pl.pallas_call(
        _prelude_body,
        out_shape=[
            jax.ShapeDtypeStruct((_APAD, n), jnp.int32),
            jax.ShapeDtypeStruct((_APAD, n), jnp.int32),
            jax.ShapeDtypeStruct((_APAD, n), jnp.float32),
            jax.ShapeDtypeStruct((_APAD, n), jnp.float32),
        ],
    )(x.T, gamma.reshape(_ACTIONS, 1), beta.reshape(_ACTIONS, 1))

    nchunk = -(-B.shape[0] // 4096)
    b_blk = pl.pallas_call(
        _convert_body,
        grid=(nchunk,),
        in_specs=[pl.BlockSpec((_EMB, 4096), lambda g: (0, g))],
        out_specs=pl.BlockSpec((1024, 128), lambda g: (g, 0)),
        out_shape=jax.ShapeDtypeStruct((nchunk * 1024, 128), jnp.float32),
    )(B.T)
    b_rm = b_blk.reshape(nchunk * 4096, _EMB)

    mesh = plsc.VectorSubcoreMesh(core_axis_name="c", subcore_axis_name="s")
    sc = pl.kernel(
        _sc_body,
        out_type=jax.ShapeDtypeStruct((_ACTIONS, _EMB, n), jnp.float32),
        mesh=mesh,
        compiler_params=pltpu.CompilerParams(
            needs_layout_passes=False, use_tc_tiling_on_sc=False),
        scratch_types=[
            pltpu.VMEM((_ACTIONS, _BLK), jnp.int32),
            pltpu.VMEM((_ACTIONS, _BLK), jnp.int32),
            pltpu.VMEM((_ACTIONS, _BLK), jnp.float32),
            pltpu.VMEM((_ACTIONS, _BLK), jnp.float32),
            pltpu.VMEM((_BLK, _EMB), jnp.float32),
            pltpu.VMEM((_BLK, _EMB), jnp.float32),
            pltpu.VMEM((_BLK, _EMB), jnp.float32),
            pltpu.VMEM((_BLK, _EMB), jnp.float32),
            pltpu.VMEM((_EMB, _BLK), jnp.float32),
            pltpu.VMEM((_EMB, _BLK), jnp.float32),
            pltpu.SemaphoreType.DMA,
            pltpu.SemaphoreType.DMA,
            pltpu.SemaphoreType.DMA,
            pltpu.SemaphoreType.DMA,
        ],
    )
    out = sc(idxlo, idxhi, wl, wh, b_rm)
    return jnp.transpose(out, (2, 0, 1))


# speedup vs baseline: 1.2546x; 1.2546x over previous
import jax
import jax.numpy as jnp
from jax import lax
from jax.experimental import pallas as pl
from jax.experimental.pallas import tpu as pltpu
from jax.experimental.pallas import tpu_sc as plsc

_ACTIONS = 100
_APAD = 104
_EMB = 32
_DELTA = 5000
_LANES = 16

_NC = 2
_NS = 16
_NW = _NC * _NS
_BLK = 128


def _permute_index(i):
    return ((i >> 12) * 4096 + (i & 1023) * 4 + ((i >> 10) & 3))


def _convert_body(bt_ref, out_ref):
    y = bt_ref[...]
    zs = [y[:, j * 1024:(j + 1) * 1024].T for j in range(4)]
    out_ref[...] = jnp.concatenate(zs, axis=1)


def _prelude_body(x_ref, gamma_ref, beta_ref, idxlo_ref, idxhi_ref,
                  wl_ref, wh_ref):
    x = x_ref[...]
    mean = jnp.mean(x, axis=1, keepdims=True)
    var = jnp.mean((x - mean) ** 2, axis=1, keepdims=True)
    xn = (x - mean) / jnp.sqrt(var + 1e-5) * gamma_ref[...] + beta_ref[...]
    xt = jnp.tanh(xn)
    xc = jnp.clip(xt, -1.0 + 1e-5, 1.0 - 1e-5)
    ind = lax.broadcasted_iota(jnp.int32, x.shape, 0)
    xl = jnp.floor(xc * _DELTA)
    xh = jnp.floor(xc * _DELTA + 1)
    xli = _permute_index(_ACTIONS * (xl.astype(jnp.int32) + _DELTA) + ind)
    xhi = _permute_index(_ACTIONS * (xh.astype(jnp.int32) + _DELTA) + ind)
    d = 1.0 / _DELTA
    wh = (xc - xl / _DELTA) / d
    wl = (xh / _DELTA - xc) / d
    pad = _APAD - _ACTIONS
    zi = jnp.zeros((pad, x.shape[1]), jnp.int32)
    zf = jnp.zeros((pad, x.shape[1]), jnp.float32)
    idxlo_ref[...] = jnp.concatenate([xli, zi], axis=0)
    idxhi_ref[...] = jnp.concatenate([xhi, zi], axis=0)
    wl_ref[...] = jnp.concatenate([wl, zf], axis=0)
    wh_ref[...] = jnp.concatenate([wh, zf], axis=0)


def _sc_body(idxlo_hbm, idxhi_hbm, wl_hbm, wh_hbm, table_hbm, out_hbm,
             il_v, ih_v, wl_v, wh_v, bl0_v, bl1_v, bh0_v, bh1_v,
             out0_v, out1_v, gsem0, gsem1, osem0, osem1):
    wid = lax.axis_index("s") * _NC + lax.axis_index("c")
    bs = wid * _BLK
    pltpu.sync_copy(idxlo_hbm.at[:_ACTIONS, pl.ds(bs, _BLK)], il_v)
    pltpu.sync_copy(idxhi_hbm.at[:_ACTIONS, pl.ds(bs, _BLK)], ih_v)
    pltpu.sync_copy(wl_hbm.at[:_ACTIONS, pl.ds(bs, _BLK)], wl_v)
    pltpu.sync_copy(wh_hbm.at[:_ACTIONS, pl.ds(bs, _BLK)], wh_v)

    bls = (bl0_v, bl1_v)
    bhs = (bh0_v, bh1_v)
    outs = (out0_v, out1_v)
    gsems = (gsem0, gsem1)
    osems = (osem0, osem1)

    def fire(a, slot):
        pltpu.async_copy(table_hbm.at[il_v.at[a]], bls[slot], gsems[slot])
        pltpu.async_copy(table_hbm.at[ih_v.at[a]], bhs[slot], gsems[slot])

    fire(0, 0)
    fire(1, 1)

    def body(i, carry):
        for slot in range(2):
            a = 2 * i + slot
            bl_v, bh_v, out_v = bls[slot], bhs[slot], outs[slot]
            pltpu.make_async_copy(
                table_hbm.at[il_v.at[a]], bl_v, gsems[slot]).wait()
            pltpu.make_async_copy(
                table_hbm.at[ih_v.at[a]], bh_v, gsems[slot]).wait()

            @pl.when(i > 0)
            def _():
                for eb in range(_EMB // 8):
                    pltpu.make_async_copy(
                        out_v.at[pl.ds(eb * 8, 8)],
                        out_hbm.at[a, eb, wid], osems[slot]).wait()

            ngrp = _BLK // _LANES
            wlgs = [wl_v[a, pl.ds(g * _LANES, _LANES)] for g in range(ngrp)]
            whgs = [wh_v[a, pl.ds(g * _LANES, _LANES)] for g in range(ngrp)]
            base = lax.iota(jnp.int32, _LANES)
            rowidxs = [base + g * _LANES for g in range(ngrp)]

            @plsc.parallel_loop(0, _EMB, unroll=4)
            def _(e):
                colidx = jnp.zeros((_LANES,), jnp.int32) + e
                for g in range(ngrp):
                    blv = plsc.load_gather(bl_v, [rowidxs[g], colidx])
                    bhv = plsc.load_gather(bh_v, [rowidxs[g], colidx])
                    out_v[e, pl.ds(g * _LANES, _LANES)] = (
                        blv * wlgs[g] + bhv * whgs[g])

            for eb in range(_EMB // 8):
                pltpu.async_copy(
                    out_v.at[pl.ds(eb * 8, 8)],
                    out_hbm.at[a, eb, wid], osems[slot])

            @pl.when(i < _ACTIONS // 2 - 1)
            def _():
                fire(a + 2, slot)
        return carry

    lax.fori_loop(0, _ACTIONS // 2, body, 0)
    for slot in range(2):
        a = _ACTIONS - 2 + slot
        for eb in range(_EMB // 8):
            pltpu.make_async_copy(
                outs[slot].at[pl.ds(eb * 8, 8)],
                out_hbm.at[a, eb, wid], osems[slot]).wait()


@jax.jit
def kernel(x, B, gamma, beta):
    n = x.shape[0]
    idxlo, idxhi, wl, wh = pl.pallas_call(
        _prelude_body,
        out_shape=[
            jax.ShapeDtypeStruct((_APAD, n), jnp.int32),
            jax.ShapeDtypeStruct((_APAD, n), jnp.int32),
            jax.ShapeDtypeStruct((_APAD, n), jnp.float32),
            jax.ShapeDtypeStruct((_APAD, n), jnp.float32),
        ],
    )(x.T, gamma.reshape(_ACTIONS, 1), beta.reshape(_ACTIONS, 1))

    nchunk = -(-B.shape[0] // 4096)
    b_blk = pl.pallas_call(
        _convert_body,
        grid=(nchunk,),
        in_specs=[pl.BlockSpec((_EMB, 4096), lambda g: (0, g))],
        out_specs=pl.BlockSpec((1024, 128), lambda g: (g, 0)),
        out_shape=jax.ShapeDtypeStruct((nchunk * 1024, 128), jnp.float32),
    )(B.T)
    b_rm = b_blk.reshape(nchunk * 4096, _EMB)

    mesh = plsc.VectorSubcoreMesh(core_axis_name="c", subcore_axis_name="s")
    sc = pl.kernel(
        _sc_body,
        out_type=jax.ShapeDtypeStruct(
            (_ACTIONS, _EMB // 8, n // _BLK, 8, _BLK), jnp.float32),
        mesh=mesh,
        compiler_params=pltpu.CompilerParams(
            needs_layout_passes=False, use_tc_tiling_on_sc=False),
        scratch_types=[
            pltpu.VMEM((_ACTIONS, _BLK), jnp.int32),
            pltpu.VMEM((_ACTIONS, _BLK), jnp.int32),
            pltpu.VMEM((_ACTIONS, _BLK), jnp.float32),
            pltpu.VMEM((_ACTIONS, _BLK), jnp.float32),
            pltpu.VMEM((_BLK, _EMB), jnp.float32),
            pltpu.VMEM((_BLK, _EMB), jnp.float32),
            pltpu.VMEM((_BLK, _EMB), jnp.float32),
            pltpu.VMEM((_BLK, _EMB), jnp.float32),
            pltpu.VMEM((_EMB, _BLK), jnp.float32),
            pltpu.VMEM((_EMB, _BLK), jnp.float32),
            pltpu.SemaphoreType.DMA,
            pltpu.SemaphoreType.DMA,
            pltpu.SemaphoreType.DMA,
            pltpu.SemaphoreType.DMA,
        ],
    )
    out = sc(idxlo, idxhi, wl, wh, b_rm)
    return jnp.transpose(out, (2, 4, 0, 1, 3)).reshape(n, _ACTIONS, _EMB)
